# baseline (speedup 1.0000x reference)
import functools

import jax
import jax.numpy as jnp
from jax import lax
from jax.experimental import pallas as pl
from jax.experimental.pallas import tpu as pltpu
from jax.experimental.pallas import tpu_sc as plsc

_RADIUS_SQ = 0.2 ** 2
_K = 32
_LANES = 16
_UNROLL = 4
_QGROUP = 4
_PREP_UNROLL = 8


def _bf16_round(x):
    u = plsc.bitcast(x, jnp.int32)
    u = (u + 0x7FFF + ((u >> 16) & 1)) & jnp.int32(-0x10000)
    return plsc.bitcast(u, jnp.float32)


def _sc_body(n_points, q_per_worker, num_cores,
             x_hbm, y_hbm, z_hbm, cent_hbm, out_hbm,
             x_v, y_v, z_v, sp_v, cent_v,
             cxb_v, cyb_v, czb_v, sc_v, ap0_v, ap1_v, ap2_v, ap3_v, out_v):
    ap_refs = (ap0_v, ap1_v, ap2_v, ap3_v)
    wid = lax.axis_index("s") * num_cores + lax.axis_index("c")
    groups_per_batch = cent_hbm.shape[1] // q_per_worker
    b = wid // groups_per_batch
    sgrp = lax.rem(wid, groups_per_batch)

    pltpu.sync_copy(x_hbm.at[b], x_v)
    pltpu.sync_copy(y_hbm.at[b], y_v)
    pltpu.sync_copy(z_hbm.at[b], z_v)
    pltpu.sync_copy(cent_hbm.at[b, pl.ds(sgrp * q_per_worker, q_per_worker)],
                    cent_v)

    def prep_centers(g, carry):
        sl = pl.ds(g * _LANES, _LANES)
        cidx = cent_v[sl]
        cx = plsc.load_gather(x_v, [cidx])
        cy = plsc.load_gather(y_v, [cidx])
        cz = plsc.load_gather(z_v, [cidx])
        sc_v[sl] = (cx * cx + cy * cy) + cz * cz
        cxb_v[sl] = _bf16_round(cx)
        cyb_v[sl] = _bf16_round(cy)
        czb_v[sl] = _bf16_round(cz)
        return carry

    lax.fori_loop(0, q_per_worker // _LANES, prep_centers, 0)

    def prep_points(g, carry):
        for u in range(_PREP_UNROLL):
            sl = pl.ds(g * (_PREP_UNROLL * _LANES) + u * _LANES, _LANES)
            xs, ys, zs = x_v[sl], y_v[sl], z_v[sl]
            sp_v[sl] = (xs * xs + ys * ys) + zs * zs
            x_v[sl] = _bf16_round(xs)
            y_v[sl] = _bf16_round(ys)
            z_v[sl] = _bf16_round(zs)
        return carry

    lax.fori_loop(0, n_points // (_PREP_UNROLL * _LANES), prep_points, 0)

    lane_iota = lax.iota(jnp.int32, _LANES)
    zeros16 = jnp.zeros((_LANES,), jnp.int32)
    two = jnp.float32(2.0)
    r2 = jnp.float32(_RADIUS_SQ)

    def do_qgroup(grp, carry):
        qbase = grp * _QGROUP
        consts = []
        for j in range(_QGROUP):
            qsplat = jnp.full((_LANES,), qbase + j, jnp.int32)
            consts.append((plsc.load_gather(cxb_v, [qsplat]),
                           plsc.load_gather(cyb_v, [qsplat]),
                           plsc.load_gather(czb_v, [qsplat]),
                           plsc.load_gather(sc_v, [qsplat])))

        def cond(state):
            i = state[0]
            notdone = state[1] < _K
            for j in range(2, _QGROUP + 1):
                notdone = jnp.logical_or(notdone, state[j] < _K)
            return jnp.logical_and(notdone, i < n_points)

        def body(state):
            i = state[0]
            offs = list(state[1:])
            for u in range(_UNROLL):
                sl = pl.ds(i + u * _LANES, _LANES)
                xs, ys, zs, sp = x_v[sl], y_v[sl], z_v[sl], sp_v[sl]
                idxv = lane_iota + (i + u * _LANES)
                for j in range(_QGROUP):
                    cxb, cyb, czb, scq = consts[j]
                    cross = (xs * cxb + ys * cyb) + zs * czb
                    d2 = (scq + sp) - two * cross
                    m = jnp.logical_and(d2 <= r2,
                                        jnp.full((_LANES,), offs[j] < _K))
                    plsc.store_compressed(ap_refs[j].at[pl.ds(offs[j], _LANES)],
                                          idxv, mask=m)
                    offs[j] = offs[j] + jnp.sum(m.astype(jnp.int32))
            return (i + _UNROLL * _LANES, *offs)

        state = lax.while_loop(cond, body,
                               (jnp.int32(0),) + (jnp.int32(0),) * _QGROUP)

        for j in range(_QGROUP):
            cnt = state[1 + j]
            first = plsc.load_gather(ap_refs[j], [zeros16])
            cntv = jnp.full((_LANES,), cnt, jnp.int32)
            for h in range(_K // _LANES):
                vals = ap_refs[j][pl.ds(h * _LANES, _LANES)]
                lane = lane_iota + h * _LANES
                out_v[pl.ds((qbase + j) * _K + h * _LANES, _LANES)] = jnp.where(
                    lane < cntv, vals, first)
        return carry

    lax.fori_loop(0, q_per_worker // _QGROUP, do_qgroup, 0)

    pltpu.sync_copy(out_v, out_hbm.at[wid])


def kernel(pos, centroids):
    B, N, _ = pos.shape
    S = centroids.shape[1]
    info = plsc.get_sparse_core_info()
    num_workers = info.num_cores * info.num_subcores
    q_per_worker = (B * S) // num_workers

    x = pos[:, :, 0]
    y = pos[:, :, 1]
    z = pos[:, :, 2]

    body = functools.partial(_sc_body, N, q_per_worker, info.num_cores)
    out = pl.kernel(
        body,
        out_type=jax.ShapeDtypeStruct((num_workers, q_per_worker * _K),
                                      jnp.int32),
        mesh=plsc.VectorSubcoreMesh(core_axis_name="c", subcore_axis_name="s"),
        compiler_params=pltpu.CompilerParams(needs_layout_passes=False),
        scratch_types=[
            pltpu.VMEM((N,), jnp.float32),
            pltpu.VMEM((N,), jnp.float32),
            pltpu.VMEM((N,), jnp.float32),
            pltpu.VMEM((N,), jnp.float32),
            pltpu.VMEM((q_per_worker,), jnp.int32),
            pltpu.VMEM((q_per_worker,), jnp.float32),
            pltpu.VMEM((q_per_worker,), jnp.float32),
            pltpu.VMEM((q_per_worker,), jnp.float32),
            pltpu.VMEM((q_per_worker,), jnp.float32),
            pltpu.VMEM((_K + _LANES,), jnp.int32),
            pltpu.VMEM((_K + _LANES,), jnp.int32),
            pltpu.VMEM((_K + _LANES,), jnp.int32),
            pltpu.VMEM((_K + _LANES,), jnp.int32),
            pltpu.VMEM((q_per_worker * _K,), jnp.int32),
        ],
    )(x, y, z, centroids)
    return out.reshape(B, S, _K)


# speedup vs baseline: 1.6915x; 1.6915x over previous
import functools

import jax
import jax.numpy as jnp
from jax import lax
from jax.experimental import pallas as pl
from jax.experimental.pallas import tpu as pltpu
from jax.experimental.pallas import tpu_sc as plsc

_RADIUS_SQ = 0.2 ** 2
_K = 32
_LANES = 16
_UNROLL = 16
_PREP_UNROLL = 8


def _bf16_round(x):
    u = plsc.bitcast(x, jnp.int32)
    u = (u + 0x7FFF + ((u >> 16) & 1)) & jnp.int32(-0x10000)
    return plsc.bitcast(u, jnp.float32)


def _popcount(m):
    return plsc.all_reduce_population_count(m)[0]


def _sc_body(n_points, q_per_worker, num_cores,
             x_hbm, y_hbm, z_hbm, cent_hbm, out_hbm,
             x_v, y_v, z_v, sp_v, cent_v,
             cxb_v, cyb_v, czb_v, sc_v, ap_v, out_v):
    wid = lax.axis_index("s") * num_cores + lax.axis_index("c")
    groups_per_batch = cent_hbm.shape[1] // q_per_worker
    b = wid // groups_per_batch
    sgrp = lax.rem(wid, groups_per_batch)

    pltpu.sync_copy(x_hbm.at[b], x_v)
    pltpu.sync_copy(y_hbm.at[b], y_v)
    pltpu.sync_copy(z_hbm.at[b], z_v)
    pltpu.sync_copy(cent_hbm.at[b, pl.ds(sgrp * q_per_worker, q_per_worker)],
                    cent_v)

    def prep_centers(g, carry):
        sl = pl.ds(g * _LANES, _LANES)
        cidx = cent_v[sl]
        cx = plsc.load_gather(x_v, [cidx])
        cy = plsc.load_gather(y_v, [cidx])
        cz = plsc.load_gather(z_v, [cidx])
        sc_v[sl] = (cx * cx + cy * cy) + cz * cz
        cxb_v[sl] = _bf16_round(cx)
        cyb_v[sl] = _bf16_round(cy)
        czb_v[sl] = _bf16_round(cz)
        return carry

    lax.fori_loop(0, q_per_worker // _LANES, prep_centers, 0)

    def prep_points(g, carry):
        for u in range(_PREP_UNROLL):
            sl = pl.ds(g * (_PREP_UNROLL * _LANES) + u * _LANES, _LANES)
            xs, ys, zs = x_v[sl], y_v[sl], z_v[sl]
            sp_v[sl] = (xs * xs + ys * ys) + zs * zs
            x_v[sl] = _bf16_round(xs)
            y_v[sl] = _bf16_round(ys)
            z_v[sl] = _bf16_round(zs)
        return carry

    lax.fori_loop(0, n_points // (_PREP_UNROLL * _LANES), prep_points, 0)

    lane_iota = lax.iota(jnp.int32, _LANES)
    zeros16 = jnp.zeros((_LANES,), jnp.int32)
    two = jnp.float32(2.0)
    r2 = jnp.float32(_RADIUS_SQ)

    def do_query(q, carry):
        qsplat = jnp.full((_LANES,), q, jnp.int32)
        cxb = plsc.load_gather(cxb_v, [qsplat])
        cyb = plsc.load_gather(cyb_v, [qsplat])
        czb = plsc.load_gather(czb_v, [qsplat])
        scq = plsc.load_gather(sc_v, [qsplat])

        def cond(state):
            i, cnt = state
            return jnp.logical_and(cnt < _K, i < n_points)

        def body(state):
            i, cnt = state
            ms = []
            for u in range(_UNROLL):
                sl = pl.ds(i + u * _LANES, _LANES)
                cross = (x_v[sl] * cxb + y_v[sl] * cyb) + z_v[sl] * czb
                d2 = (scq + sp_v[sl]) - two * cross
                ms.append(d2 <= r2)
            sums = [_popcount(m) for m in ms]
            off = cnt
            for u in range(_UNROLL):
                plsc.store_compressed(ap_v.at[pl.ds(off, _LANES)],
                                      lane_iota + (i + u * _LANES),
                                      mask=ms[u])
                off = off + sums[u]
            return i + _UNROLL * _LANES, off

        _, cnt = lax.while_loop(cond, body, (jnp.int32(0), jnp.int32(0)))

        first = plsc.load_gather(ap_v, [zeros16])
        cntv = jnp.full((_LANES,), cnt, jnp.int32)
        for h in range(_K // _LANES):
            vals = ap_v[pl.ds(h * _LANES, _LANES)]
            lane = lane_iota + h * _LANES
            out_v[pl.ds(q * _K + h * _LANES, _LANES)] = jnp.where(
                lane < cntv, vals, first)
        return carry

    lax.fori_loop(0, q_per_worker, do_query, 0)

    pltpu.sync_copy(out_v, out_hbm.at[wid])


def kernel(pos, centroids):
    B, N, _ = pos.shape
    S = centroids.shape[1]
    info = plsc.get_sparse_core_info()
    num_workers = info.num_cores * info.num_subcores
    q_per_worker = (B * S) // num_workers

    x = pos[:, :, 0]
    y = pos[:, :, 1]
    z = pos[:, :, 2]

    body = functools.partial(_sc_body, N, q_per_worker, info.num_cores)
    out = pl.kernel(
        body,
        out_type=jax.ShapeDtypeStruct((num_workers, q_per_worker * _K),
                                      jnp.int32),
        mesh=plsc.VectorSubcoreMesh(core_axis_name="c", subcore_axis_name="s"),
        compiler_params=pltpu.CompilerParams(needs_layout_passes=False),
        scratch_types=[
            pltpu.VMEM((N,), jnp.float32),
            pltpu.VMEM((N,), jnp.float32),
            pltpu.VMEM((N,), jnp.float32),
            pltpu.VMEM((N,), jnp.float32),
            pltpu.VMEM((q_per_worker,), jnp.int32),
            pltpu.VMEM((q_per_worker,), jnp.float32),
            pltpu.VMEM((q_per_worker,), jnp.float32),
            pltpu.VMEM((q_per_worker,), jnp.float32),
            pltpu.VMEM((q_per_worker,), jnp.float32),
            pltpu.VMEM((_K + _UNROLL * _LANES,), jnp.int32),
            pltpu.VMEM((q_per_worker * _K,), jnp.int32),
        ],
    )(x, y, z, centroids)
    return out.reshape(B, S, _K)
